# baseline (speedup 1.0000x reference)
import functools

import jax
import jax.numpy as jnp
from jax.experimental import pallas as pl
from jax.experimental.pallas import tpu as pltpu

_N = 10000
_D = 128
_BM = 200
_BA = 2000

_HI = jax.lax.Precision.HIGHEST


def _dot(a, b, precision=None):
    return jax.lax.dot_general(a, b, (((1,), (0,)), ((), ())),
                               precision=precision,
                               preferred_element_type=jnp.float32)


def _stage_a_body(x_ref, w1_ref, sw1_ref, b1_ref, s1_ref, p1_ref):
    x = x_ref[...]
    s1 = _dot(x, w1_ref[...], _HI) + b1_ref[...]
    s1_ref[...] = s1.astype(jnp.bfloat16)
    p1_ref[...] = _dot(x, sw1_ref[...], _HI)


def _stage_b_body(g_ref, s1_ref, p1_ref, w2_ref, sw2_ref, b2_ref,
                  s2_ref, p2_ref):
    g = g_ref[...].astype(jnp.bfloat16)
    h = jnp.maximum(_dot(g, s1_ref[...]) + p1_ref[...], 0.0)
    s2 = _dot(h, w2_ref[...], _HI) + b2_ref[...]
    s2_ref[...] = s2.astype(jnp.bfloat16)
    p2_ref[...] = _dot(h, sw2_ref[...], _HI)


def _stage_c_body(g_ref, s2_ref, p2_ref, o_ref):
    g = g_ref[...].astype(jnp.bfloat16)
    o_ref[...] = _dot(g, s2_ref[...]) + p2_ref[...]


@jax.jit
def kernel(input, G, W1, SW1, b1, W2, SW2, b2):
    x = input
    b1r = b1.reshape(1, _D)
    b2r = b2.reshape(1, _D)

    full_w = pl.BlockSpec((_D, _D), lambda i: (0, 0))
    full_b = pl.BlockSpec((1, _D), lambda i: (0, 0))
    full_s = pl.BlockSpec((_N, _D), lambda i: (0, 0))
    parallel = pltpu.CompilerParams(dimension_semantics=("parallel",))

    s1, p1 = pl.pallas_call(
        _stage_a_body,
        grid=(_N // _BA,),
        in_specs=[pl.BlockSpec((_BA, _D), lambda i: (i, 0)),
                  full_w, full_w, full_b],
        out_specs=[pl.BlockSpec((_BA, _D), lambda i: (i, 0)),
                   pl.BlockSpec((_BA, _D), lambda i: (i, 0))],
        out_shape=[jax.ShapeDtypeStruct((_N, _D), jnp.bfloat16),
                   jax.ShapeDtypeStruct((_N, _D), jnp.float32)],
        compiler_params=parallel,
    )(x, W1, SW1, b1r)

    s2, p2 = pl.pallas_call(
        _stage_b_body,
        grid=(_N // _BM,),
        in_specs=[pl.BlockSpec((_BM, _N), lambda i: (i, 0)),
                  full_s,
                  pl.BlockSpec((_BM, _D), lambda i: (i, 0)),
                  full_w, full_w, full_b],
        out_specs=[pl.BlockSpec((_BM, _D), lambda i: (i, 0)),
                   pl.BlockSpec((_BM, _D), lambda i: (i, 0))],
        out_shape=[jax.ShapeDtypeStruct((_N, _D), jnp.bfloat16),
                   jax.ShapeDtypeStruct((_N, _D), jnp.float32)],
        compiler_params=parallel,
    )(G, s1, p1, W2, SW2, b2r)

    out = pl.pallas_call(
        _stage_c_body,
        grid=(_N // _BM,),
        in_specs=[pl.BlockSpec((_BM, _N), lambda i: (i, 0)),
                  full_s,
                  pl.BlockSpec((_BM, _D), lambda i: (i, 0))],
        out_specs=pl.BlockSpec((_BM, _D), lambda i: (i, 0)),
        out_shape=jax.ShapeDtypeStruct((_N, _D), jnp.float32),
        compiler_params=parallel,
    )(G, s2, p2)
    return out


# speedup vs baseline: 1.0053x; 1.0053x over previous
import jax
import jax.numpy as jnp
from jax.experimental import pallas as pl
from jax.experimental.pallas import tpu as pltpu

_N = 10000
_D = 128
_BM = 400
_BA = 2000


def _dot(a, b):
    return jax.lax.dot_general(a, b, (((1,), (0,)), ((), ())),
                               preferred_element_type=jnp.float32)


def _stage_a_body(x_ref, w_ref, b_ref, s1_ref, p1_ref):
    r = _dot(x_ref[...].astype(jnp.bfloat16), w_ref[...]) + b_ref[...]
    s1_ref[...] = r[:, :_D].astype(jnp.bfloat16)
    p1_ref[...] = r[:, _D:]


def _stage_b_body(g_ref, s1_ref, p1_ref, w_ref, b_ref, s2_ref, p2_ref):
    g = g_ref[...].astype(jnp.bfloat16)
    h = jnp.maximum(_dot(g, s1_ref[...]) + p1_ref[...], 0.0)
    r = _dot(h.astype(jnp.bfloat16), w_ref[...]) + b_ref[...]
    s2_ref[...] = r[:, :_D].astype(jnp.bfloat16)
    p2_ref[...] = r[:, _D:]


def _stage_c_body(g_ref, s2_ref, p2_ref, o_ref):
    g = g_ref[...].astype(jnp.bfloat16)
    o_ref[...] = _dot(g, s2_ref[...]) + p2_ref[...]


@jax.jit
def kernel(input, G, W1, SW1, b1, W2, SW2, b2):
    x = input
    w1c = jnp.concatenate([W1, SW1], axis=1).astype(jnp.bfloat16)
    w2c = jnp.concatenate([W2, SW2], axis=1).astype(jnp.bfloat16)
    zeros = jnp.zeros((1, _D), jnp.float32)
    b1c = jnp.concatenate([b1.reshape(1, _D), zeros], axis=1)
    b2c = jnp.concatenate([b2.reshape(1, _D), zeros], axis=1)

    full_w = pl.BlockSpec((_D, 2 * _D), lambda i: (0, 0))
    full_b = pl.BlockSpec((1, 2 * _D), lambda i: (0, 0))
    full_s = pl.BlockSpec((_N, _D), lambda i: (0, 0))
    parallel = pltpu.CompilerParams(dimension_semantics=("parallel",))

    s1, p1 = pl.pallas_call(
        _stage_a_body,
        grid=(_N // _BA,),
        in_specs=[pl.BlockSpec((_BA, _D), lambda i: (i, 0)),
                  full_w, full_b],
        out_specs=[pl.BlockSpec((_BA, _D), lambda i: (i, 0)),
                   pl.BlockSpec((_BA, _D), lambda i: (i, 0))],
        out_shape=[jax.ShapeDtypeStruct((_N, _D), jnp.bfloat16),
                   jax.ShapeDtypeStruct((_N, _D), jnp.float32)],
        compiler_params=parallel,
    )(x, w1c, b1c)

    s2, p2 = pl.pallas_call(
        _stage_b_body,
        grid=(_N // _BM,),
        in_specs=[pl.BlockSpec((_BM, _N), lambda i: (i, 0)),
                  full_s,
                  pl.BlockSpec((_BM, _D), lambda i: (i, 0)),
                  full_w, full_b],
        out_specs=[pl.BlockSpec((_BM, _D), lambda i: (i, 0)),
                   pl.BlockSpec((_BM, _D), lambda i: (i, 0))],
        out_shape=[jax.ShapeDtypeStruct((_N, _D), jnp.bfloat16),
                   jax.ShapeDtypeStruct((_N, _D), jnp.float32)],
        compiler_params=parallel,
    )(G, s1, p1, w2c, b2c)

    out = pl.pallas_call(
        _stage_c_body,
        grid=(_N // _BM,),
        in_specs=[pl.BlockSpec((_BM, _N), lambda i: (i, 0)),
                  full_s,
                  pl.BlockSpec((_BM, _D), lambda i: (i, 0))],
        out_specs=pl.BlockSpec((_BM, _D), lambda i: (i, 0)),
        out_shape=jax.ShapeDtypeStruct((_N, _D), jnp.float32),
        compiler_params=parallel,
    )(G, s2, p2)
    return out


# speedup vs baseline: 1.0428x; 1.0373x over previous
import jax
import jax.numpy as jnp
from jax.experimental import pallas as pl
from jax.experimental.pallas import tpu as pltpu

_N = 10000
_D = 128
_BM = 400
_BA = 2000
_BF = jnp.bfloat16


def _dot(a, b):
    return jax.lax.dot_general(a, b, (((1,), (0,)), ((), ())),
                               preferred_element_type=jnp.float32)


def _stage_a_body(x_ref, w1_ref, sw1_ref, b1_ref, s1_ref, p1_ref):
    x = x_ref[...].astype(_BF)
    s1_ref[...] = (_dot(x, w1_ref[...].astype(_BF)) + b1_ref[...]).astype(_BF)
    p1_ref[...] = _dot(x, sw1_ref[...].astype(_BF)).astype(_BF)


def _stage_b_body(g_ref, s1_ref, p1_ref, w2_ref, sw2_ref, b2_ref,
                  s2_ref, p2_ref):
    g = g_ref[...].astype(_BF)
    h = jnp.maximum(_dot(g, s1_ref[...]) + p1_ref[...].astype(jnp.float32),
                    0.0).astype(_BF)
    s2_ref[...] = (_dot(h, w2_ref[...].astype(_BF)) + b2_ref[...]).astype(_BF)
    p2_ref[...] = _dot(h, sw2_ref[...].astype(_BF)).astype(_BF)


def _stage_c_body(g_ref, s2_ref, p2_ref, o_ref):
    g = g_ref[...].astype(_BF)
    o_ref[...] = _dot(g, s2_ref[...]) + p2_ref[...].astype(jnp.float32)


@jax.jit
def kernel(input, G, W1, SW1, b1, W2, SW2, b2):
    x = input
    b1r = b1.reshape(1, _D)
    b2r = b2.reshape(1, _D)

    full_w = pl.BlockSpec((_D, _D), lambda i: (0, 0))
    full_b = pl.BlockSpec((1, _D), lambda i: (0, 0))
    full_s = pl.BlockSpec((_N, _D), lambda i: (0, 0))
    parallel = pltpu.CompilerParams(dimension_semantics=("parallel",))

    s1, p1 = pl.pallas_call(
        _stage_a_body,
        grid=(_N // _BA,),
        in_specs=[pl.BlockSpec((_BA, _D), lambda i: (i, 0)),
                  full_w, full_w, full_b],
        out_specs=[pl.BlockSpec((_BA, _D), lambda i: (i, 0)),
                   pl.BlockSpec((_BA, _D), lambda i: (i, 0))],
        out_shape=[jax.ShapeDtypeStruct((_N, _D), _BF),
                   jax.ShapeDtypeStruct((_N, _D), _BF)],
        compiler_params=parallel,
    )(x, W1, SW1, b1r)

    s2, p2 = pl.pallas_call(
        _stage_b_body,
        grid=(_N // _BM,),
        in_specs=[pl.BlockSpec((_BM, _N), lambda i: (i, 0)),
                  full_s,
                  pl.BlockSpec((_BM, _D), lambda i: (i, 0)),
                  full_w, full_w, full_b],
        out_specs=[pl.BlockSpec((_BM, _D), lambda i: (i, 0)),
                   pl.BlockSpec((_BM, _D), lambda i: (i, 0))],
        out_shape=[jax.ShapeDtypeStruct((_N, _D), _BF),
                   jax.ShapeDtypeStruct((_N, _D), _BF)],
        compiler_params=parallel,
    )(G, s1, p1, W2, SW2, b2r)

    out = pl.pallas_call(
        _stage_c_body,
        grid=(_N // _BM,),
        in_specs=[pl.BlockSpec((_BM, _N), lambda i: (i, 0)),
                  full_s,
                  pl.BlockSpec((_BM, _D), lambda i: (i, 0))],
        out_specs=pl.BlockSpec((_BM, _D), lambda i: (i, 0)),
        out_shape=jax.ShapeDtypeStruct((_N, _D), jnp.float32),
        compiler_params=parallel,
    )(G, s2, p2)
    return out


# speedup vs baseline: 1.1195x; 1.0736x over previous
import jax
import jax.numpy as jnp
from jax.experimental import pallas as pl
from jax.experimental.pallas import tpu as pltpu

_N = 10000
_D = 128
_BM = 400
_NB = _N // _BM
_BF = jnp.bfloat16


def _dot(a, b):
    return jax.lax.dot_general(a, b, (((1,), (0,)), ((), ())),
                               preferred_element_type=jnp.float32)


def _body(x_ref, g_ref, w1_ref, sw1_ref, b1_ref, w2_ref, sw2_ref, b2_ref,
          o_ref, s1_scr, p1_scr, s2_scr, p2_scr):
    i = pl.program_id(0)

    @pl.when(i == 0)
    def _stage_a():
        x = x_ref[...].astype(_BF)
        s1_scr[...] = (_dot(x, w1_ref[...].astype(_BF))
                       + b1_ref[...]).astype(_BF)
        p1_scr[...] = _dot(x, sw1_ref[...].astype(_BF)).astype(_BF)

    @pl.when((i >= 1) & (i <= _NB))
    def _pass_1():
        r = (i - 1) * _BM
        g = g_ref[...].astype(_BF)
        h = jnp.maximum(
            _dot(g, s1_scr[...])
            + p1_scr[pl.ds(r, _BM), :].astype(jnp.float32), 0.0).astype(_BF)
        s2_scr[pl.ds(r, _BM), :] = (_dot(h, w2_ref[...].astype(_BF))
                                    + b2_ref[...]).astype(_BF)
        p2_scr[pl.ds(r, _BM), :] = _dot(h, sw2_ref[...].astype(_BF)).astype(_BF)

    @pl.when(i > _NB)
    def _pass_2():
        r = (i - 1 - _NB) * _BM
        g = g_ref[...].astype(_BF)
        o_ref[...] = (_dot(g, s2_scr[...])
                      + p2_scr[pl.ds(r, _BM), :].astype(jnp.float32))


@jax.jit
def kernel(input, G, W1, SW1, b1, W2, SW2, b2):
    x = input
    b1r = b1.reshape(1, _D)
    b2r = b2.reshape(1, _D)

    inv = lambda i: (0, 0)
    return pl.pallas_call(
        _body,
        grid=(2 * _NB + 1,),
        in_specs=[
            pl.BlockSpec((_N, _D), inv),
            pl.BlockSpec((_BM, _N),
                         lambda i: (jnp.maximum(i - 1, 0) % _NB, 0)),
            pl.BlockSpec((_D, _D), inv),
            pl.BlockSpec((_D, _D), inv),
            pl.BlockSpec((1, _D), inv),
            pl.BlockSpec((_D, _D), inv),
            pl.BlockSpec((_D, _D), inv),
            pl.BlockSpec((1, _D), inv),
        ],
        out_specs=pl.BlockSpec((_BM, _D),
                               lambda i: (jnp.maximum(i - 1 - _NB, 0), 0)),
        out_shape=jax.ShapeDtypeStruct((_N, _D), jnp.float32),
        scratch_shapes=[
            pltpu.VMEM((_N, _D), _BF),
            pltpu.VMEM((_N, _D), _BF),
            pltpu.VMEM((_N, _D), _BF),
            pltpu.VMEM((_N, _D), _BF),
        ],
        compiler_params=pltpu.CompilerParams(
            dimension_semantics=("arbitrary",)),
    )(x, G, W1, SW1, b1r, W2, SW2, b2r)
